# Initial kernel scaffold; baseline (speedup 1.0000x reference)
import jax
import jax.numpy as jnp
from jax.experimental import pallas as pl


def kernel(X, row_pointers, column_index, degrees, W, b):
    raise NotImplementedError("write your pallas kernel here")


# speedup vs baseline: 27.0689x; 27.0689x over previous
import functools

import jax
import jax.numpy as jnp
from jax import lax
from jax.experimental import pallas as pl
from jax.experimental.pallas import tpu as pltpu
from jax.experimental.pallas import tpu_sc as plsc

_NC = 2
_NS = 16
_L = 16
_C = 128


def _tc_linear(X, W, b, degrees):
    n, d_in = X.shape
    d_out = W.shape[0]
    blk = 400
    assert n % blk == 0

    def body(x_ref, w_ref, b_ref, d_ref, y_ref):
        xp = lax.dot_general(x_ref[...], w_ref[...], (((1,), (1,)), ((), ())),
                             preferred_element_type=jnp.float32)
        y_ref[...] = (xp + b_ref[...]) * d_ref[...]

    return pl.pallas_call(
        body,
        grid=(n // blk,),
        in_specs=[
            pl.BlockSpec((blk, d_in), lambda i: (i, 0)),
            pl.BlockSpec((d_out, d_in), lambda i: (0, 0)),
            pl.BlockSpec((1, d_out), lambda i: (0, 0)),
            pl.BlockSpec((blk, 1), lambda i: (i, 0)),
        ],
        out_specs=pl.BlockSpec((blk, d_out), lambda i: (i, 0)),
        out_shape=jax.ShapeDtypeStruct((n, d_out), jnp.float32),
    )(X, W, b.reshape(1, d_out), degrees.reshape(n, 1))


def _sc_spmm(y, rp_pad, col, deg_pad, *, n_pad, rpt, d):
    dump = _NS * rpt
    bs_iters = max(1, (rpt - 1).bit_length())
    mesh = plsc.VectorSubcoreMesh(core_axis_name="c", subcore_axis_name="s")

    @functools.partial(
        pl.kernel,
        out_type=jax.ShapeDtypeStruct((n_pad, d), jnp.float32),
        mesh=mesh,
        compiler_params=pltpu.CompilerParams(needs_layout_passes=False),
        scratch_types=[
            pltpu.VMEM((rpt + 8,), jnp.int32),
            pltpu.VMEM((rpt,), jnp.float32),
            pltpu.VMEM((_C,), jnp.int32),
            pltpu.VMEM((_C,), jnp.int32),
            pltpu.VMEM((_C, d), jnp.float32),
            pltpu.VMEM((rpt, d), jnp.float32),
            pltpu.VMEM_SHARED((_NS * rpt + 8, d), jnp.float32),
        ],
    )
    def spmm(y_hbm, rp_hbm, col_hbm, deg_hbm, out_hbm,
             rp_v, d_v, colv, rid_v, rows_v, out_v, acc_sh):
        c = lax.axis_index("c")
        s = lax.axis_index("s")
        wid = s * _NC + c
        r0 = wid * rpt
        acc_base = s * rpt

        def zrow(r, carry):
            for j in range(d // _L):
                out_v[r, pl.ds(j * _L, _L)] = jnp.zeros((_L,), jnp.float32)
            return carry
        lax.fori_loop(0, rpt, zrow, 0)
        pltpu.sync_copy(out_v, acc_sh.at[pl.ds(acc_base, rpt)])

        pltpu.sync_copy(rp_hbm.at[pl.ds(r0, rpt + 8)], rp_v)
        pltpu.sync_copy(deg_hbm.at[pl.ds(r0, rpt)], d_v)
        e0 = rp_v[pl.ds(0, _L)][0]
        e1 = rp_v[pl.ds(rpt - 8, _L)][8]
        k0 = e0 // _C
        k1 = (e1 + _C - 1) // _C

        lanes = jnp.arange(_L, dtype=jnp.int32)

        def chunk(k, carry):
            base = k * _C
            pltpu.sync_copy(col_hbm.at[pl.ds(base, _C)], colv)
            pltpu.sync_copy(y_hbm.at[colv], rows_v)
            ec_hi = jnp.maximum(e0, e1 - 1)
            for g in range(_C // _L):
                e = base + g * _L + lanes
                valid = (e >= e0) & (e < e1)
                ec = jnp.clip(e, e0, ec_hi)
                lo = jnp.zeros((_L,), jnp.int32)
                hi = jnp.full((_L,), rpt, jnp.int32)

                for _ in range(bs_iters):
                    mid = (lo + hi) // 2
                    cle = plsc.load_gather(rp_v, [mid]) <= ec
                    lo = jnp.where(cle, mid, lo)
                    hi = jnp.where(cle, hi, mid)
                rid_v[pl.ds(g * _L, _L)] = jnp.where(valid, acc_base + lo, dump)
            pltpu.sync_copy(rows_v, acc_sh.at[rid_v], add=True)
            return carry
        lax.fori_loop(k0, k1, chunk, 0)

        pltpu.sync_copy(acc_sh.at[pl.ds(acc_base, rpt)], out_v)
        def scale(t, carry):
            dvec = d_v[pl.ds(t * _L, _L)]
            for i in range(_L):
                dr = dvec[i]
                r = t * _L + i
                for j in range(d // _L):
                    out_v[r, pl.ds(j * _L, _L)] = out_v[r, pl.ds(j * _L, _L)] * dr
            return carry
        lax.fori_loop(0, rpt // _L, scale, 0)
        pltpu.sync_copy(out_v, out_hbm.at[pl.ds(r0, rpt)])

    return spmm(y, rp_pad, col, deg_pad)


def kernel(X, row_pointers, column_index, degrees, W, b):
    n, _ = X.shape
    d_out = W.shape[0]
    e = column_index.shape[0]
    nw = _NC * _NS
    rpt = ((-(-n // nw) + _L - 1) // _L) * _L
    n_pad = nw * rpt

    y = _tc_linear(X, W, b, degrees)

    rp_pad = jnp.concatenate([
        row_pointers,
        jnp.full((n_pad + 8 - (n + 1),), jnp.int32(e)),
    ])
    deg_pad = jnp.concatenate([degrees, jnp.zeros((n_pad - n,), jnp.float32)])
    pad_e = (-e) % _C
    col = column_index
    if pad_e:
        col = jnp.concatenate([col, jnp.zeros((pad_e,), jnp.int32)])

    out = _sc_spmm(y, rp_pad, col, deg_pad, n_pad=n_pad, rpt=rpt, d=d_out)
    return out[:n]


# speedup vs baseline: 49.3773x; 1.8241x over previous
import functools

import jax
import jax.numpy as jnp
from jax import lax
from jax.experimental import pallas as pl
from jax.experimental.pallas import tpu as pltpu
from jax.experimental.pallas import tpu_sc as plsc

_NC = 2
_NS = 16
_L = 16
_C = 128


def _tc_linear(X, W, b, degrees):
    n, d_in = X.shape
    d_out = W.shape[0]
    blk = 400
    assert n % blk == 0

    def body(x_ref, w_ref, b_ref, d_ref, y_ref):
        xp = lax.dot_general(x_ref[...], w_ref[...], (((1,), (1,)), ((), ())),
                             preferred_element_type=jnp.float32)
        y_ref[...] = (xp + b_ref[...]) * d_ref[...]

    return pl.pallas_call(
        body,
        grid=(n // blk,),
        in_specs=[
            pl.BlockSpec((blk, d_in), lambda i: (i, 0)),
            pl.BlockSpec((d_out, d_in), lambda i: (0, 0)),
            pl.BlockSpec((1, d_out), lambda i: (0, 0)),
            pl.BlockSpec((blk, 1), lambda i: (i, 0)),
        ],
        out_specs=pl.BlockSpec((blk, d_out), lambda i: (i, 0)),
        out_shape=jax.ShapeDtypeStruct((n, d_out), jnp.float32),
    )(X, W, b.reshape(1, d_out), degrees.reshape(n, 1))


def _sc_spmm(y, rp_pad, col, deg_pad, *, n_pad, rpt, d):
    dump = _NS * rpt
    bs_iters = max(1, (rpt - 1).bit_length())
    mesh = plsc.VectorSubcoreMesh(core_axis_name="c", subcore_axis_name="s")

    @functools.partial(
        pl.kernel,
        out_type=jax.ShapeDtypeStruct((n_pad, d), jnp.float32),
        mesh=mesh,
        compiler_params=pltpu.CompilerParams(needs_layout_passes=False),
        scratch_types=[
            pltpu.VMEM((rpt + 8,), jnp.int32),
            pltpu.VMEM((rpt,), jnp.float32),
            pltpu.VMEM((_C,), jnp.int32),
            pltpu.VMEM((_C,), jnp.int32),
            pltpu.VMEM((_C,), jnp.int32),
            pltpu.VMEM((_C,), jnp.int32),
            pltpu.VMEM((_C, d), jnp.float32),
            pltpu.VMEM((_C, d), jnp.float32),
            pltpu.VMEM((rpt, d), jnp.float32),
            pltpu.VMEM_SHARED((_NS * rpt + 8, d), jnp.float32),
            pltpu.SemaphoreType.DMA,
            pltpu.SemaphoreType.DMA,
            pltpu.SemaphoreType.DMA,
            pltpu.SemaphoreType.DMA,
            pltpu.SemaphoreType.DMA,
            pltpu.SemaphoreType.DMA,
        ],
    )
    def spmm(y_hbm, rp_hbm, col_hbm, deg_hbm, out_hbm,
             rp_v, d_v, colv0, colv1, rid0, rid1, rows0, rows1, out_v, acc_sh,
             gs0, gs1, ss0, ss1, cs0, cs1):
        c = lax.axis_index("c")
        s = lax.axis_index("s")
        wid = s * _NC + c
        r0 = wid * rpt
        acc_base = s * rpt

        def zrow(r, carry):
            for j in range(d // _L):
                out_v[r, pl.ds(j * _L, _L)] = jnp.zeros((_L,), jnp.float32)
            return carry
        lax.fori_loop(0, rpt, zrow, 0)
        pltpu.sync_copy(out_v, acc_sh.at[pl.ds(acc_base, rpt)])

        pltpu.sync_copy(rp_hbm.at[pl.ds(r0, rpt + 8)], rp_v)
        pltpu.sync_copy(deg_hbm.at[pl.ds(r0, rpt)], d_v)
        e0 = rp_v[pl.ds(0, _L)][0]
        e1 = rp_v[pl.ds(rpt - 8, _L)][8]
        k0 = e0 // _C
        k1 = (e1 + _C - 1) // _C

        lanes = jnp.arange(_L, dtype=jnp.int32)
        ec_hi = jnp.maximum(e0, e1 - 1)
        nk = k1 - k0
        slots = ((colv0, rid0, rows0, gs0, ss0, cs0),
                 (colv1, rid1, rows1, gs1, ss1, cs1))

        def col_desc(k, cv, cs):
            return pltpu.make_async_copy(col_hbm.at[pl.ds(k * _C, _C)], cv, cs)

        def gat_desc(cv, rv, gs):
            return pltpu.make_async_copy(y_hbm.at[cv], rv, gs)

        def sct_desc(rv, iv, ss):
            return pltpu.make_async_copy(rv, acc_sh.at[iv], ss)

        def compute_rid(k, rid_ref):
            base = k * _C
            for g in range(_C // _L):
                e = base + g * _L + lanes
                valid = (e >= e0) & (e < e1)
                ec = jnp.clip(e, e0, ec_hi)
                lo = jnp.zeros((_L,), jnp.int32)
                hi = jnp.full((_L,), rpt, jnp.int32)
                for _ in range(bs_iters):
                    mid = (lo + hi) // 2
                    cle = plsc.load_gather(rp_v, [mid]) <= ec
                    lo = jnp.where(cle, mid, lo)
                    hi = jnp.where(cle, hi, mid)
                rid_ref[pl.ds(g * _L, _L)] = jnp.where(valid, acc_base + lo, dump)

        @pl.when(nk > 0)
        def _prologue():
            pltpu.sync_copy(col_hbm.at[pl.ds(k0 * _C, _C)], colv0)
            pltpu.async_copy(y_hbm.at[colv0], rows0, gs0)

        @pl.when(nk > 1)
        def _prologue2():
            col_desc(k0 + 1, colv1, cs1).start()

        def pair(io, carry):
            for b in range(2):
                cv_c, rid_c, rw_c, gs_c, ss_c, cs_c = slots[b]
                cv_n, rid_n, rw_n, gs_n, ss_n, cs_n = slots[1 - b]
                k = k0 + 2 * io + b

                @pl.when(k < k1)
                def _step():
                    @pl.when(k + 1 < k1)
                    def _launch_next():
                        @pl.when(k > k0)
                        def _drain_prev_scatter():
                            sct_desc(rw_n, rid_n, ss_n).wait()
                        col_desc(k + 1, cv_n, cs_n).wait()
                        gat_desc(cv_n, rw_n, gs_n).start()

                    compute_rid(k, rid_c)
                    gat_desc(cv_c, rw_c, gs_c).wait()

                    @pl.when(k + 2 < k1)
                    def _prefetch_col():
                        col_desc(k + 2, cv_c, cs_c).start()

                    pltpu.async_copy(rw_c, acc_sh.at[rid_c], ss_c, add=True)
            return carry
        lax.fori_loop(0, (nk + 1) // 2, pair, 0)

        @pl.when((nk >= 1) & (((nk - 1) & 1) == 0))
        def _d0():
            sct_desc(rows0, rid0, ss0).wait()

        @pl.when((nk >= 1) & (((nk - 1) & 1) == 1))
        def _d1():
            sct_desc(rows1, rid1, ss1).wait()

        @pl.when((nk >= 2) & (((nk - 2) & 1) == 0))
        def _d2():
            sct_desc(rows0, rid0, ss0).wait()

        @pl.when((nk >= 2) & (((nk - 2) & 1) == 1))
        def _d3():
            sct_desc(rows1, rid1, ss1).wait()

        pltpu.sync_copy(acc_sh.at[pl.ds(acc_base, rpt)], out_v)
        def scale(t, carry):
            dvec = d_v[pl.ds(t * _L, _L)]
            for i in range(_L):
                dr = dvec[i]
                r = t * _L + i
                for j in range(d // _L):
                    out_v[r, pl.ds(j * _L, _L)] = out_v[r, pl.ds(j * _L, _L)] * dr
            return carry
        lax.fori_loop(0, rpt // _L, scale, 0)
        pltpu.sync_copy(out_v, out_hbm.at[pl.ds(r0, rpt)])

    return spmm(y, rp_pad, col, deg_pad)


def kernel(X, row_pointers, column_index, degrees, W, b):
    n, _ = X.shape
    d_out = W.shape[0]
    e = column_index.shape[0]
    nw = _NC * _NS
    rpt = ((-(-n // nw) + _L - 1) // _L) * _L
    n_pad = nw * rpt

    y = _tc_linear(X, W, b, degrees)

    rp_pad = jnp.concatenate([
        row_pointers,
        jnp.full((n_pad + 8 - (n + 1),), jnp.int32(e)),
    ])
    deg_pad = jnp.concatenate([degrees, jnp.zeros((n_pad - n,), jnp.float32)])
    pad_e = (-e) % _C
    col = column_index
    if pad_e:
        col = jnp.concatenate([col, jnp.zeros((pad_e,), jnp.int32)])

    out = _sc_spmm(y, rp_pad, col, deg_pad, n_pad=n_pad, rpt=rpt, d=d_out)
    return out[:n]
